# baseline (speedup 1.0000x reference)
import functools

import jax
import jax.numpy as jnp
from jax import lax
from jax.experimental import pallas as pl
from jax.experimental.pallas import tpu as pltpu
from jax.experimental.pallas import tpu_sc as plsc

N = 100000
CIN = 16
COUT = 16
KVOL = 27

NW = 32
BN = 1024
NPAD = 102400
C_PER_W = NPAD // NW
CHUNK = 128
NCH = C_PER_W // CHUNK
ROWS128 = NPAD // CHUNK
ZERO_ROW = N


def _idx_body(kmap_ref, mask_ref, idx_ref):
    i = pl.program_id(0)
    n = i * C_PER_W + lax.broadcasted_iota(jnp.int32, (KVOL, C_PER_W), 1)
    valid = (mask_ref[...] != 0) & (n < N)
    v = jnp.where(valid, kmap_ref[...], ZERO_ROW)
    idx_ref[...] = v.reshape(KVOL, 1, NCH, CHUNK)


def _build_idx(kmap32, mask32):
    return pl.pallas_call(
        _idx_body,
        grid=(NW,),
        in_specs=[
            pl.BlockSpec((KVOL, C_PER_W), lambda i: (0, i)),
            pl.BlockSpec((KVOL, C_PER_W), lambda i: (0, i)),
        ],
        out_specs=pl.BlockSpec((KVOL, 1, NCH, CHUNK), lambda i: (0, i, 0, 0)),
        out_shape=jax.ShapeDtypeStruct((KVOL, NW, NCH, CHUNK), jnp.int32),
    )(kmap32, mask32)


def _sc_gather_body(feats_hbm, idx_hbm, g2_hbm, idx_v, buf, sem):
    info = plsc.get_sparse_core_info()
    nc = info.num_cores
    wid = lax.axis_index("s") * nc + lax.axis_index("c")
    base = wid * C_PER_W

    def kbody(k, _):
        pltpu.sync_copy(idx_hbm.at[k, wid], idx_v)

        def cbody(c5, _):
            handles = []
            for j in range(5):
                c = c5 * 5 + j
                handles.append(
                    pltpu.async_copy(
                        feats_hbm.at[idx_v.at[c]],
                        buf.at[pl.ds(c * CHUNK, CHUNK)],
                        sem,
                    )
                )
            for h in handles:
                h.wait()
            return 0

        lax.fori_loop(0, NCH // 5, cbody, 0)
        pltpu.sync_copy(buf, g2_hbm.at[k, pl.ds(base, C_PER_W)])
        return 0

    lax.fori_loop(0, KVOL, kbody, 0)


def _sc_gather(feats_pad, idx3):
    mesh = plsc.VectorSubcoreMesh(core_axis_name="c", subcore_axis_name="s")
    fn = functools.partial(
        pl.kernel,
        mesh=mesh,
        out_type=jax.ShapeDtypeStruct((KVOL, NPAD, CIN), jnp.float32),
        scratch_types=[
            pltpu.VMEM((NCH, CHUNK), jnp.int32),
            pltpu.VMEM((C_PER_W, CIN), jnp.float32),
            pltpu.SemaphoreType.DMA,
        ],
        compiler_params=pltpu.CompilerParams(use_tc_tiling_on_sc=False),
    )(_sc_gather_body)
    return fn(feats_pad, idx3)


def _mm_body(g2_ref, w_ref, b_ref, o_ref):
    k = pl.program_id(1)
    part = jnp.dot(g2_ref[0], w_ref[0], preferred_element_type=jnp.float32)

    @pl.when(k == 0)
    def _():
        o_ref[...] = part + b_ref[...]

    @pl.when(k > 0)
    def _():
        o_ref[...] += part


def _matmul(g2, weight, bias2):
    grid = (NPAD // BN, KVOL)
    return pl.pallas_call(
        _mm_body,
        grid=grid,
        in_specs=[
            pl.BlockSpec((1, BN, CIN), lambda i, k: (k, i, 0)),
            pl.BlockSpec((1, CIN, COUT), lambda i, k: (k, 0, 0)),
            pl.BlockSpec((1, COUT), lambda i, k: (0, 0)),
        ],
        out_specs=pl.BlockSpec((BN, COUT), lambda i, k: (i, 0)),
        out_shape=jax.ShapeDtypeStruct((NPAD, COUT), jnp.float32),
        compiler_params=pltpu.CompilerParams(
            dimension_semantics=("arbitrary", "arbitrary"),
        ),
    )(g2, weight, bias2)


def kernel(feats, kmap, mask, weight, bias):
    kmap32 = kmap.astype(jnp.int32)
    mask32 = mask.astype(jnp.int32)
    feats_pad = jnp.concatenate(
        [feats, jnp.zeros((8, CIN), dtype=feats.dtype)], axis=0
    )
    idx3 = _build_idx(kmap32, mask32)
    g2 = _sc_gather(feats_pad, idx3)
    out_full = _matmul(g2, weight, bias.reshape(1, COUT))
    return out_full[:N]


# speedup vs baseline: 1.0047x; 1.0047x over previous
import functools

import jax
import jax.numpy as jnp
from jax import lax
from jax.experimental import pallas as pl
from jax.experimental.pallas import tpu as pltpu
from jax.experimental.pallas import tpu_sc as plsc

N = 100000
CIN = 16
COUT = 16
KVOL = 27

NW = 32
BN = 1024
NPAD = 102400
C_PER_W = NPAD // NW
CHUNK = 128
NCH = C_PER_W // CHUNK
ROWS128 = NPAD // CHUNK
ZERO_ROW = N


def _idx_body(kmap_ref, mask_ref, idx_ref):
    i = pl.program_id(0)
    n = i * C_PER_W + lax.broadcasted_iota(jnp.int32, (KVOL, C_PER_W), 1)
    valid = (mask_ref[...] != 0) & (n < N)
    v = jnp.where(valid, kmap_ref[...], ZERO_ROW)
    idx_ref[...] = v.reshape(1, KVOL, NCH, CHUNK)


def _build_idx(kmap32, mask32):
    return pl.pallas_call(
        _idx_body,
        grid=(NW,),
        in_specs=[
            pl.BlockSpec((KVOL, C_PER_W), lambda i: (0, i)),
            pl.BlockSpec((KVOL, C_PER_W), lambda i: (0, i)),
        ],
        out_specs=pl.BlockSpec((1, KVOL, NCH, CHUNK), lambda i: (i, 0, 0, 0)),
        out_shape=jax.ShapeDtypeStruct((NW, KVOL, NCH, CHUNK), jnp.int32),
    )(kmap32, mask32)


TOT = KVOL * NCH
RING = 8
DEPTH = 4


def _sc_gather_body(feats_hbm, idx_hbm, g2_hbm, idx_all, gbuf, sem_g, sem_w):
    info = plsc.get_sparse_core_info()
    nc = info.num_cores
    wid = lax.axis_index("s") * nc + lax.axis_index("c")
    base = wid * C_PER_W

    pltpu.sync_copy(idx_hbm.at[wid], idx_all)

    def fire(t):
        k = t // NCH
        c = t % NCH
        pltpu.async_copy(
            feats_hbm.at[idx_all.at[k, c]], gbuf.at[t % RING], sem_g
        )

    for t in range(DEPTH):
        fire(t)

    def body(t, _):
        slot = t % RING
        pltpu.make_async_copy(
            feats_hbm.at[pl.ds(0, CHUNK)], gbuf.at[slot], sem_g
        ).wait()
        k = t // NCH
        c = t % NCH
        pltpu.async_copy(
            gbuf.at[slot], g2_hbm.at[k, pl.ds(base + c * CHUNK, CHUNK)], sem_w
        )
        t_next = t + DEPTH

        @pl.when(t_next < TOT)
        def _():
            @pl.when(t_next >= RING)
            def _():
                pltpu.make_async_copy(
                    gbuf.at[slot], g2_hbm.at[0, pl.ds(base, CHUNK)], sem_w
                ).wait()

            fire(t_next)

        return 0

    lax.fori_loop(0, TOT, body, 0)

    def ebody(i, _):
        pltpu.make_async_copy(
            gbuf.at[0], g2_hbm.at[0, pl.ds(base, CHUNK)], sem_w
        ).wait()
        return 0

    lax.fori_loop(0, RING, ebody, 0)


def _sc_gather(feats_pad, idx3):
    mesh = plsc.VectorSubcoreMesh(core_axis_name="c", subcore_axis_name="s")
    fn = functools.partial(
        pl.kernel,
        mesh=mesh,
        out_type=jax.ShapeDtypeStruct((KVOL, NPAD, CIN), jnp.float32),
        scratch_types=[
            pltpu.VMEM((KVOL, NCH, CHUNK), jnp.int32),
            pltpu.VMEM((RING, CHUNK, CIN), jnp.float32),
            pltpu.SemaphoreType.DMA,
            pltpu.SemaphoreType.DMA,
        ],
        compiler_params=pltpu.CompilerParams(use_tc_tiling_on_sc=False),
    )(_sc_gather_body)
    return fn(feats_pad, idx3)


def _mm_body(g2_ref, w_ref, b_ref, o_ref):
    k = pl.program_id(1)
    part = jnp.dot(g2_ref[0], w_ref[0], preferred_element_type=jnp.float32)

    @pl.when(k == 0)
    def _():
        o_ref[...] = part + b_ref[...]

    @pl.when(k > 0)
    def _():
        o_ref[...] += part


def _matmul(g2, weight, bias2):
    grid = (NPAD // BN, KVOL)
    return pl.pallas_call(
        _mm_body,
        grid=grid,
        in_specs=[
            pl.BlockSpec((1, BN, CIN), lambda i, k: (k, i, 0)),
            pl.BlockSpec((1, CIN, COUT), lambda i, k: (k, 0, 0)),
            pl.BlockSpec((1, COUT), lambda i, k: (0, 0)),
        ],
        out_specs=pl.BlockSpec((BN, COUT), lambda i, k: (i, 0)),
        out_shape=jax.ShapeDtypeStruct((NPAD, COUT), jnp.float32),
        compiler_params=pltpu.CompilerParams(
            dimension_semantics=("arbitrary", "arbitrary"),
        ),
    )(g2, weight, bias2)


def kernel(feats, kmap, mask, weight, bias):
    kmap32 = kmap.astype(jnp.int32)
    mask32 = mask.astype(jnp.int32)
    feats_pad = jnp.concatenate(
        [feats, jnp.zeros((8, CIN), dtype=feats.dtype)], axis=0
    )
    idx3 = _build_idx(kmap32, mask32)
    g2 = _sc_gather(feats_pad, idx3)
    out_full = _matmul(g2, weight, bias.reshape(1, COUT))
    return out_full[:N]


# speedup vs baseline: 2.9155x; 2.9018x over previous
import functools

import jax
import jax.numpy as jnp
from jax import lax
from jax.experimental import pallas as pl
from jax.experimental.pallas import tpu as pltpu
from jax.experimental.pallas import tpu_sc as plsc

N = 100000
CIN = 16
COUT = 16
KVOL = 27

NW = 32
BN = 1024
NPAD = 102400
C_PER_W = NPAD // NW
CHUNK = 128
NCH = C_PER_W // CHUNK
ROWS128 = NPAD // CHUNK
ZERO_ROW = N


def _idx_body(kmap_ref, mask_ref, idx_ref):
    i = pl.program_id(0)
    n = i * C_PER_W + lax.broadcasted_iota(jnp.int32, (KVOL, C_PER_W), 1)
    valid = (mask_ref[...] != 0) & (n < N)
    v = jnp.where(valid, kmap_ref[...], ZERO_ROW)
    idx_ref[...] = v.reshape(1, KVOL, NCH, CHUNK)


def _build_idx(kmap32, mask32):
    return pl.pallas_call(
        _idx_body,
        grid=(NW,),
        in_specs=[
            pl.BlockSpec((KVOL, C_PER_W), lambda i: (0, i)),
            pl.BlockSpec((KVOL, C_PER_W), lambda i: (0, i)),
        ],
        out_specs=pl.BlockSpec((1, KVOL, NCH, CHUNK), lambda i: (i, 0, 0, 0)),
        out_shape=jax.ShapeDtypeStruct((NW, KVOL, NCH, CHUNK), jnp.int32),
    )(kmap32, mask32)


TOT = KVOL * NCH
RI = 8
RG = 6
DI = 4
DG = 3


def _sc_gather_body(feats_hbm, idx_hbm, g2_hbm, feats_sp, idxr, gbuf,
                    sem_i, sem_g, sem_w):
    info = plsc.get_sparse_core_info()
    nc = info.num_cores
    wid = lax.axis_index("s") * nc + lax.axis_index("c")
    base = wid * C_PER_W

    @pl.when(lax.axis_index("s") == 0)
    def _():
        pltpu.sync_copy(feats_hbm, feats_sp)

    def fire_idx(t):
        k = t // NCH
        c = t % NCH
        pltpu.async_copy(idx_hbm.at[wid, k, c], idxr.at[t % RI], sem_i)

    def drain_idx(t):
        pltpu.make_async_copy(
            idx_hbm.at[0, 0, 0], idxr.at[t % RI], sem_i
        ).wait()

    def fire_gather(t):
        pltpu.async_copy(
            feats_sp.at[idxr.at[t % RI]], gbuf.at[t % RG], sem_g
        )

    def drain_gather(t):
        pltpu.make_async_copy(
            feats_hbm.at[pl.ds(0, CHUNK)], gbuf.at[t % RG], sem_g
        ).wait()

    def fire_writeout(t):
        k = t // NCH
        c = t % NCH
        pltpu.async_copy(
            gbuf.at[t % RG], g2_hbm.at[k, pl.ds(base + c * CHUNK, CHUNK)],
            sem_w,
        )

    def drain_writeout():
        pltpu.make_async_copy(
            gbuf.at[0], g2_hbm.at[0, pl.ds(base, CHUNK)], sem_w
        ).wait()

    for t in range(DG + DI):
        fire_idx(t)
    plsc.subcore_barrier()
    for t in range(DG):
        drain_idx(t)
        fire_gather(t)

    def body(t, _):
        drain_gather(t)
        fire_writeout(t)
        tg = t + DG

        @pl.when(tg < TOT)
        def _():
            @pl.when(tg >= RG)
            def _():
                drain_writeout()

            drain_idx(tg)
            fire_gather(tg)
            ti = tg + DI

            @pl.when(ti < TOT)
            def _():
                fire_idx(ti)

        return 0

    lax.fori_loop(0, TOT, body, 0)

    def ebody(i, _):
        drain_writeout()
        return 0

    lax.fori_loop(0, min(RG, TOT), ebody, 0)


def _sc_gather(feats_pad, idx3):
    mesh = plsc.VectorSubcoreMesh(core_axis_name="c", subcore_axis_name="s")
    fn = functools.partial(
        pl.kernel,
        mesh=mesh,
        out_type=jax.ShapeDtypeStruct((KVOL, NPAD, CIN), jnp.float32),
        scratch_types=[
            pltpu.VMEM_SHARED((N + 8, CIN), jnp.float32),
            pltpu.VMEM((RI, CHUNK), jnp.int32),
            pltpu.VMEM((RG, CHUNK, CIN), jnp.float32),
            pltpu.SemaphoreType.DMA,
            pltpu.SemaphoreType.DMA,
            pltpu.SemaphoreType.DMA,
        ],
        compiler_params=pltpu.CompilerParams(use_tc_tiling_on_sc=False),
    )(_sc_gather_body)
    return fn(feats_pad, idx3)


def _mm_body(g2_ref, w_ref, b_ref, o_ref):
    k = pl.program_id(1)
    part = jnp.dot(g2_ref[0], w_ref[0], preferred_element_type=jnp.float32)

    @pl.when(k == 0)
    def _():
        o_ref[...] = part + b_ref[...]

    @pl.when(k > 0)
    def _():
        o_ref[...] += part


def _matmul(g2, weight, bias2):
    grid = (NPAD // BN, KVOL)
    return pl.pallas_call(
        _mm_body,
        grid=grid,
        in_specs=[
            pl.BlockSpec((1, BN, CIN), lambda i, k: (k, i, 0)),
            pl.BlockSpec((1, CIN, COUT), lambda i, k: (k, 0, 0)),
            pl.BlockSpec((1, COUT), lambda i, k: (0, 0)),
        ],
        out_specs=pl.BlockSpec((BN, COUT), lambda i, k: (i, 0)),
        out_shape=jax.ShapeDtypeStruct((NPAD, COUT), jnp.float32),
        compiler_params=pltpu.CompilerParams(
            dimension_semantics=("arbitrary", "arbitrary"),
        ),
    )(g2, weight, bias2)


def kernel(feats, kmap, mask, weight, bias):
    kmap32 = kmap.astype(jnp.int32)
    mask32 = mask.astype(jnp.int32)
    feats_pad = jnp.concatenate(
        [feats, jnp.zeros((8, CIN), dtype=feats.dtype)], axis=0
    )
    idx3 = _build_idx(kmap32, mask32)
    g2 = _sc_gather(feats_pad, idx3)
    out_full = _matmul(g2, weight, bias.reshape(1, COUT))
    return out_full[:N]


# speedup vs baseline: 9.7067x; 3.3293x over previous
import functools

import jax
import jax.numpy as jnp
from jax import lax
from jax.experimental import pallas as pl
from jax.experimental.pallas import tpu as pltpu
from jax.experimental.pallas import tpu_sc as plsc

N = 100000
CIN = 16
COUT = 16
KVOL = 27

NW = 32
BN = 1024
NPAD = 102400
C_PER_W = NPAD // NW
CHUNK = 128
NCH = C_PER_W // CHUNK
ROWS128 = NPAD // CHUNK
ZERO_ROW = N


def _idx_body(kmap_ref, mask_ref, idx_ref):
    i = pl.program_id(0)
    n = i * C_PER_W + lax.broadcasted_iota(jnp.int32, (KVOL, C_PER_W), 1)
    valid = (mask_ref[...] != 0) & (n < N)
    v = jnp.where(valid, kmap_ref[...], ZERO_ROW)
    idx_ref[...] = v.reshape(1, KVOL, NCH, CHUNK)


def _build_idx(kmap32, mask32):
    return pl.pallas_call(
        _idx_body,
        grid=(NW,),
        in_specs=[
            pl.BlockSpec((KVOL, C_PER_W), lambda i: (0, i)),
            pl.BlockSpec((KVOL, C_PER_W), lambda i: (0, i)),
        ],
        out_specs=pl.BlockSpec((1, KVOL, NCH, CHUNK), lambda i: (i, 0, 0, 0)),
        out_shape=jax.ShapeDtypeStruct((NW, KVOL, NCH, CHUNK), jnp.int32),
    )(kmap32, mask32)


TOT = KVOL * NCH
RI = 8
RG = 6
DI = 4
DG = 3


def _sc_gather_body(feats_hbm, idx_hbm, g2_hbm, feats_sp, idxr, gbuf,
                    sem_i, sem_g, sem_w):
    info = plsc.get_sparse_core_info()
    nc = info.num_cores
    wid = lax.axis_index("s") * nc + lax.axis_index("c")
    base = wid * C_PER_W

    @pl.when(lax.axis_index("s") == 0)
    def _():
        pltpu.sync_copy(feats_hbm, feats_sp)

    def fire_idx(t):
        k = t // NCH
        c = t % NCH
        pltpu.async_copy(idx_hbm.at[wid, k, c], idxr.at[t % RI], sem_i)

    def drain_idx(t):
        pltpu.make_async_copy(
            idx_hbm.at[0, 0, 0], idxr.at[t % RI], sem_i
        ).wait()

    def fire_gather(t):
        pltpu.async_copy(
            feats_sp.at[idxr.at[t % RI]], gbuf.at[t % RG], sem_g
        )

    def drain_gather(t):
        pltpu.make_async_copy(
            feats_hbm.at[pl.ds(0, CHUNK)], gbuf.at[t % RG], sem_g
        ).wait()

    def fire_writeout(t):
        k = t // NCH
        c = t % NCH
        pltpu.async_copy(
            gbuf.at[t % RG],
            g2_hbm.at[pl.ds(base + c * CHUNK, CHUNK), pl.ds(k * CIN, CIN)],
            sem_w,
        )

    def drain_writeout():
        pltpu.make_async_copy(
            gbuf.at[0], g2_hbm.at[pl.ds(base, CHUNK), pl.ds(0, CIN)], sem_w
        ).wait()

    for t in range(DG + DI):
        fire_idx(t)
    plsc.subcore_barrier()
    for t in range(DG):
        drain_idx(t)
        fire_gather(t)

    def body(t, _):
        drain_gather(t)
        fire_writeout(t)
        tg = t + DG

        @pl.when(tg < TOT)
        def _():
            @pl.when(tg >= RG)
            def _():
                drain_writeout()

            drain_idx(tg)
            fire_gather(tg)
            ti = tg + DI

            @pl.when(ti < TOT)
            def _():
                fire_idx(ti)

        return 0

    lax.fori_loop(0, TOT, body, 0)

    def ebody(i, _):
        drain_writeout()
        return 0

    lax.fori_loop(0, min(RG, TOT), ebody, 0)


def _sc_gather(feats_pad, idx3):
    mesh = plsc.VectorSubcoreMesh(core_axis_name="c", subcore_axis_name="s")
    fn = functools.partial(
        pl.kernel,
        mesh=mesh,
        out_type=jax.ShapeDtypeStruct((NPAD, KVOL * CIN), jnp.float32),
        scratch_types=[
            pltpu.VMEM_SHARED((N + 8, CIN), jnp.float32),
            pltpu.VMEM((RI, CHUNK), jnp.int32),
            pltpu.VMEM((RG, CHUNK, CIN), jnp.float32),
            pltpu.SemaphoreType.DMA,
            pltpu.SemaphoreType.DMA,
            pltpu.SemaphoreType.DMA,
        ],
        compiler_params=pltpu.CompilerParams(use_tc_tiling_on_sc=False),
    )(_sc_gather_body)
    return fn(feats_pad, idx3)


BM = 2048


def _mm_body(g_ref, w_ref, b_ref, o_ref):
    o_ref[...] = (
        jnp.dot(g_ref[...], w_ref[...], preferred_element_type=jnp.float32)
        + b_ref[...]
    )


def _matmul(g2, wflat, bias2):
    return pl.pallas_call(
        _mm_body,
        grid=(NPAD // BM,),
        in_specs=[
            pl.BlockSpec((BM, KVOL * CIN), lambda i: (i, 0)),
            pl.BlockSpec((KVOL * CIN, COUT), lambda i: (0, 0)),
            pl.BlockSpec((1, COUT), lambda i: (0, 0)),
        ],
        out_specs=pl.BlockSpec((BM, COUT), lambda i: (i, 0)),
        out_shape=jax.ShapeDtypeStruct((NPAD, COUT), jnp.float32),
        compiler_params=pltpu.CompilerParams(
            dimension_semantics=("arbitrary",),
        ),
    )(g2, wflat, bias2)


def kernel(feats, kmap, mask, weight, bias):
    kmap32 = kmap.astype(jnp.int32)
    mask32 = mask.astype(jnp.int32)
    feats_pad = jnp.concatenate(
        [feats, jnp.zeros((8, CIN), dtype=feats.dtype)], axis=0
    )
    idx3 = _build_idx(kmap32, mask32)
    g2 = _sc_gather(feats_pad, idx3)
    wflat = weight.reshape(KVOL * CIN, COUT)
    out_full = _matmul(g2, wflat, bias.reshape(1, COUT))
    return out_full[:N]
